# speedup vs baseline: 464.9276x; 464.9276x over previous
import jax
import jax.numpy as jnp
from jax.experimental import pallas as pl
from jax.experimental.pallas import tpu as pltpu

_ROWS, _COLS = 64, 32768
_BLOCK_ROWS = 8


def _topk_tie_kernel(x_ref, out_ref, m_ref, cnt_ref):
    i = pl.program_id(0)
    blk = x_ref[...]
    bm = jnp.max(blk)
    bc = jnp.sum((blk == bm).astype(jnp.int32))

    @pl.when(i == 0)
    def _init():
        m_ref[0] = bm
        cnt_ref[0] = bc

    @pl.when(i > 0)
    def _acc():
        m = m_ref[0]
        c = cnt_ref[0]
        new_m = jnp.maximum(m, bm)
        new_c = jnp.where(
            bm > m, bc, jnp.where(bm == m, c + bc, c)
        )
        m_ref[0] = new_m
        cnt_ref[0] = new_c

    @pl.when(i == pl.num_programs(0) - 1)
    def _emit():
        out_ref[0, 0] = (cnt_ref[0] > 1).astype(jnp.int32)


def kernel(x):
    out = pl.pallas_call(
        _topk_tie_kernel,
        grid=(_ROWS // _BLOCK_ROWS,),
        in_specs=[
            pl.BlockSpec((_BLOCK_ROWS, _COLS), lambda i: (i, 0)),
        ],
        out_specs=pl.BlockSpec(memory_space=pltpu.SMEM),
        out_shape=jax.ShapeDtypeStruct((1, 1), jnp.int32),
        scratch_shapes=[
            pltpu.SMEM((1,), jnp.float32),
            pltpu.SMEM((1,), jnp.int32),
        ],
    )(x)
    return out.reshape(()).astype(jnp.bool_)
